# baseline (speedup 1.0000x reference)
import functools

import jax
import jax.numpy as jnp
from jax import lax
from jax.experimental import pallas as pl
from jax.experimental.pallas import tpu as pltpu
from jax.experimental.pallas import tpu_sc as plsc

N = 10000
E = 320000
D = 128

NC = 2
NS = 16
NW = NC * NS
CHUNK = 128
EP = 327680
SCHUNK = EP // (NS * CHUNK)
W = 16
NWIN = SCHUNK // W
NB = 4
NH = 5000
NHPAD = 5120
SRPT = NHPAD // NS

_mesh = plsc.VectorSubcoreMesh(core_axis_name="c", subcore_axis_name="s")


def _zero_slice(buf, acc, base, nrows):
    for k in range(nrows // CHUNK):
        pltpu.sync_copy(buf, acc.at[pl.ds(base + k * CHUNK, CHUNK)])
    rem = nrows % CHUNK
    if rem:
        pltpu.sync_copy(buf.at[pl.ds(0, rem)],
                        acc.at[pl.ds(base + nrows - rem, rem)])


@functools.partial(
    pl.kernel,
    out_type=jax.ShapeDtypeStruct((NC, NHPAD, D), jnp.float32),
    mesh=_mesh,
    scratch_types=[
        pltpu.VMEM((SCHUNK, CHUNK), jnp.int32),
        pltpu.VMEM((CHUNK, D), jnp.float32),
        pltpu.VMEM_SHARED((NHPAD, D), jnp.float32),
    ],
)
def _deg_kernel(dst3, out, dst_v, buf, dacc):
    c = lax.axis_index("c")
    s = lax.axis_index("s")
    pltpu.sync_copy(dst3.at[s], dst_v)

    def fill(val):
        def row(i, carry):
            for k in range(D // 16):
                buf[i, pl.ds(k * 16, 16)] = jnp.full((16,), val, jnp.float32)
            return carry
        lax.fori_loop(0, CHUNK, row, 0)

    fill(0.0)
    base = s * SRPT
    _zero_slice(buf, dacc, base, SRPT)

    lo = c * NH

    def remap(j, carry):
        for k in range(CHUNK // 16):
            v = dst_v[j, pl.ds(k * 16, 16)]
            lc = v - lo
            ok = (lc >= 0) & (lc < NH)
            dst_v[j, pl.ds(k * 16, 16)] = jnp.where(ok, lc, NH)
        return carry
    lax.fori_loop(0, SCHUNK, remap, 0)

    fill(1.0)
    plsc.subcore_barrier()

    def chunk(j, carry):
        pltpu.sync_copy(buf, dacc.at[dst_v.at[j]], add=True)
        return carry
    lax.fori_loop(0, SCHUNK, chunk, 0)
    plsc.subcore_barrier()
    pltpu.sync_copy(dacc.at[pl.ds(base, SRPT)], out.at[c, pl.ds(base, SRPT)])


@functools.partial(
    pl.kernel,
    out_type=jax.ShapeDtypeStruct((NC, NHPAD, D), jnp.float32),
    mesh=_mesh,
    scratch_types=[
        pltpu.VMEM((2, W, CHUNK), jnp.int32),
        pltpu.VMEM((2, W, CHUNK), jnp.int32),
        [pltpu.VMEM((CHUNK, D), jnp.float32)] * NB,
        [pltpu.SemaphoreType.DMA] * NB,
        [pltpu.SemaphoreType.DMA] * NB,
        pltpu.SemaphoreType.DMA,
        pltpu.VMEM_SHARED((NHPAD, D), jnp.float32),
    ],
)
def _scatter_kernel(g, src4, dst4, out, src_w, dst_w, bufs, gsem, ssem,
                    isem, acc):
    c = lax.axis_index("c")
    s = lax.axis_index("s")
    def zrow(i, carry):
        for k in range(D // 16):
            bufs[0][i, pl.ds(k * 16, 16)] = jnp.zeros((16,), jnp.float32)
        return carry
    lax.fori_loop(0, CHUNK, zrow, 0)
    base = s * SRPT
    _zero_slice(bufs[0], acc, base, SRPT)

    lo = c * NH

    def remap_win(p):
        for rr in range(W):
            for k in range(CHUNK // 16):
                v = dst_w[p, rr, pl.ds(k * 16, 16)]
                lc = v - lo
                ok = (lc >= 0) & (lc < NH)
                dst_w[p, rr, pl.ds(k * 16, 16)] = jnp.where(ok, lc, NH)

    pltpu.sync_copy(src4.at[s, 0], src_w.at[0])
    pltpu.sync_copy(dst4.at[s, 0], dst_w.at[0])
    remap_win(0)
    plsc.subcore_barrier()

    def fire(idx_row, b):
        pltpu.async_copy(g.at[idx_row], bufs[b], gsem[b])

    def wait_g(b):
        pltpu.make_async_copy(g.at[pl.ds(0, CHUNK)], bufs[b], gsem[b]).wait()

    def wait_s(b):
        pltpu.make_async_copy(bufs[b], acc.at[dst_w.at[0, 0]], ssem[b]).wait()

    def wait_i():
        pltpu.make_async_copy(src4.at[s, 0], src_w.at[0], isem).wait()
        pltpu.make_async_copy(dst4.at[s, 0], dst_w.at[0], isem).wait()

    fire(src_w.at[0, 0], 0)
    fire(src_w.at[0, 1], 1)

    def win(w, carry):
        q = lax.rem(w, 2)
        qn = lax.rem(w + 1, 2)
        for r in range(W):
            j = w * W + r
            b = r % NB
            bn = (r + 2) % NB

            @pl.when(j >= 2)
            def _():
                wait_s(bn)

            if r == 1:
                @pl.when(w + 1 < NWIN)
                def _():
                    pltpu.async_copy(src4.at[s, w + 1], src_w.at[qn], isem)
                    pltpu.async_copy(dst4.at[s, w + 1], dst_w.at[qn], isem)

            if r == W - 2:
                @pl.when(w + 1 < NWIN)
                def _():
                    wait_i()
                    remap_win(qn)

            @pl.when(j + 2 < SCHUNK)
            def _():
                if r < W - 2:
                    fire(src_w.at[q, r + 2], bn)
                else:
                    fire(src_w.at[qn, r + 2 - W], bn)

            wait_g(b)
            pltpu.async_copy(bufs[b], acc.at[dst_w.at[q, r]], ssem[b],
                             add=True)
        return carry

    lax.fori_loop(0, NWIN, win, 0)
    wait_s((SCHUNK - 2) % NB)
    wait_s((SCHUNK - 1) % NB)
    plsc.subcore_barrier()
    pltpu.sync_copy(acc.at[pl.ds(base, SRPT)], out.at[c, pl.ds(base, SRPT)])


R = 1000


def _dinv_of(dp_ref):
    deg = dp_ref[0, :, 0] + 1.0
    return lax.rsqrt(deg)


def _tc1_body(x_ref, w_ref, dp_ref, g_ref, t_ref):
    dinv = _dinv_of(dp_ref)
    t = jnp.dot(x_ref[...], w_ref[...], preferred_element_type=jnp.float32)
    t_ref[...] = t
    g_ref[...] = t * dinv[:, None]


def _tc2_body(s_ref, dp_ref, t1_ref, w_ref, b_ref, t2_ref, g2_ref):
    dinv = _dinv_of(dp_ref)
    h = (s_ref[0] * dinv[:, None]
         + t1_ref[...] * (dinv * dinv)[:, None] + b_ref[...])
    t2 = jnp.dot(h, w_ref[...], preferred_element_type=jnp.float32)
    t2_ref[...] = t2
    g2_ref[...] = t2 * dinv[:, None]


def _tc3_body(s_ref, dp_ref, t2_ref, b_ref, out_ref):
    dinv = _dinv_of(dp_ref)
    out_ref[...] = (s_ref[0] * dinv[:, None]
                    + t2_ref[...] * (dinv * dinv)[:, None] + b_ref[...])


_row_spec = pl.BlockSpec((R, D), lambda i: (i, 0))
_w_spec = pl.BlockSpec((D, D), lambda i: (0, 0))
_s_spec = pl.BlockSpec((1, R, D), lambda i: (i // (NH // R), i % (NH // R), 0))
_b_spec = pl.BlockSpec((1, D), lambda i: (0, 0))

_tc1 = pl.pallas_call(
    _tc1_body,
    grid=(N // R,),
    in_specs=[_row_spec, _w_spec, _s_spec],
    out_specs=[_row_spec, _row_spec],
    out_shape=[jax.ShapeDtypeStruct((N, D), jnp.float32)] * 2,
)

_tc2 = pl.pallas_call(
    _tc2_body,
    grid=(N // R,),
    in_specs=[_s_spec, _s_spec, _row_spec, _w_spec, _b_spec],
    out_specs=[_row_spec, _row_spec],
    out_shape=[jax.ShapeDtypeStruct((N, D), jnp.float32)] * 2,
)

_tc3 = pl.pallas_call(
    _tc3_body,
    grid=(N // R,),
    in_specs=[_s_spec, _s_spec, _row_spec, _b_spec],
    out_specs=_row_spec,
    out_shape=jax.ShapeDtypeStruct((N, D), jnp.float32),
)


def kernel(x, adj, W1, b1, W2, b2):
    src = adj[0].astype(jnp.int32)
    dst = adj[1].astype(jnp.int32)
    pad = EP - E
    src_p = jnp.concatenate([src, jnp.zeros((pad,), jnp.int32)])
    dst_p = jnp.concatenate([dst, jnp.full((pad,), N, jnp.int32)])
    src4 = src_p.reshape(NS, NWIN, W, CHUNK)
    dst4 = dst_p.reshape(NS, NWIN, W, CHUNK)
    dst3 = dst_p.reshape(NS, SCHUNK, CHUNK)

    dp = _deg_kernel(dst3)
    g1, t1 = _tc1(x, W1, dp)
    s1 = _scatter_kernel(g1, src4, dst4)
    t2, g2 = _tc2(s1, dp, t1, W2, b1.reshape(1, D))
    s2 = _scatter_kernel(g2, src4, dst4)
    out = _tc3(s2, dp, t2, b2.reshape(1, D))
    return out


# speedup vs baseline: 1.6671x; 1.6671x over previous
import functools

import jax
import jax.numpy as jnp
from jax import lax
from jax.experimental import pallas as pl
from jax.experimental.pallas import tpu as pltpu
from jax.experimental.pallas import tpu_sc as plsc

N = 10000
E = 320000
D = 128

NC = 2
NS = 16
NW = NC * NS
CHUNK = 128
EP = 327680
TCH = EP // (NW * CHUNK)
W = 8
NWIN = TCH // W
NPAD = 10112
RPT = NPAD // NS

_mesh = plsc.VectorSubcoreMesh(core_axis_name="c", subcore_axis_name="s")


def _zero_slice(buf, acc, base, nrows):
    for k in range(nrows // CHUNK):
        pltpu.sync_copy(buf, acc.at[pl.ds(base + k * CHUNK, CHUNK)])
    rem = nrows % CHUNK
    if rem:
        pltpu.sync_copy(buf.at[pl.ds(0, rem)],
                        acc.at[pl.ds(base + nrows - rem, rem)])


def _fill(buf, val):
    def row(i, carry):
        for k in range(D // 16):
            buf[i, pl.ds(k * 16, 16)] = jnp.full((16,), val, jnp.float32)
        return carry
    lax.fori_loop(0, CHUNK, row, 0)


@functools.partial(
    pl.kernel,
    out_type=jax.ShapeDtypeStruct((NC, NPAD, D), jnp.float32),
    mesh=_mesh,
    scratch_types=[
        pltpu.VMEM((TCH, CHUNK), jnp.int32),
        pltpu.VMEM((CHUNK, D), jnp.float32),
        pltpu.VMEM_SHARED((NPAD, D), jnp.float32),
    ],
)
def _deg_kernel(dst3, out, dst_v, buf, dacc):
    c = lax.axis_index("c")
    s = lax.axis_index("s")
    w = c * NS + s
    pltpu.sync_copy(dst3.at[w], dst_v)
    _fill(buf, 0.0)
    base = s * RPT
    _zero_slice(buf, dacc, base, RPT)
    _fill(buf, 1.0)
    plsc.subcore_barrier()

    def chunk(j, carry):
        pltpu.sync_copy(buf, dacc.at[dst_v.at[j]], add=True)
        return carry
    lax.fori_loop(0, TCH, chunk, 0)
    plsc.subcore_barrier()
    pltpu.sync_copy(dacc.at[pl.ds(base, RPT)], out.at[c, pl.ds(base, RPT)])


@functools.partial(
    pl.kernel,
    out_type=jax.ShapeDtypeStruct((NC, NPAD, D), jnp.float32),
    mesh=_mesh,
    scratch_types=[
        pltpu.VMEM((2, W, CHUNK), jnp.int32),
        pltpu.VMEM((2, W, CHUNK), jnp.int32),
        [pltpu.VMEM((CHUNK, D), jnp.float32)] * 2,
        [pltpu.SemaphoreType.DMA] * 2,
        [pltpu.SemaphoreType.DMA] * 2,
        pltpu.SemaphoreType.DMA,
        pltpu.VMEM_SHARED((NPAD, D), jnp.float32),
    ],
)
def _scatter_kernel(g, src4, dst4, out, src_w, dst_w, bufs, gsem, ssem,
                    isem, acc):
    c = lax.axis_index("c")
    s = lax.axis_index("s")
    wid = c * NS + s
    _fill(bufs[0], 0.0)
    base = s * RPT
    _zero_slice(bufs[0], acc, base, RPT)

    pltpu.sync_copy(src4.at[wid, 0], src_w.at[0])
    pltpu.sync_copy(dst4.at[wid, 0], dst_w.at[0])
    plsc.subcore_barrier()

    def fire(idx_row, b):
        pltpu.async_copy(g.at[idx_row], bufs[b], gsem[b])

    def wait_g(b):
        pltpu.make_async_copy(g.at[pl.ds(0, CHUNK)], bufs[b], gsem[b]).wait()

    def wait_s(b):
        pltpu.make_async_copy(bufs[b], acc.at[dst_w.at[0, 0]], ssem[b]).wait()

    def wait_i():
        pltpu.make_async_copy(src4.at[wid, 0], src_w.at[0], isem).wait()
        pltpu.make_async_copy(dst4.at[wid, 0], dst_w.at[0], isem).wait()

    fire(src_w.at[0, 0], 0)

    def win(w, carry):
        q = lax.rem(w, 2)
        qn = lax.rem(w + 1, 2)
        for r in range(W):
            j = w * W + r
            b = r % 2
            bn = (r + 1) % 2

            @pl.when(j >= 1)
            def _():
                wait_s(bn)

            if r == 1:
                @pl.when(w + 1 < NWIN)
                def _():
                    pltpu.async_copy(src4.at[wid, w + 1], src_w.at[qn], isem)
                    pltpu.async_copy(dst4.at[wid, w + 1], dst_w.at[qn], isem)

            if r == W - 2:
                @pl.when(w + 1 < NWIN)
                def _():
                    wait_i()

            @pl.when(j + 1 < TCH)
            def _():
                if r < W - 1:
                    fire(src_w.at[q, r + 1], bn)
                else:
                    fire(src_w.at[qn, 0], bn)

            wait_g(b)
            pltpu.async_copy(bufs[b], acc.at[dst_w.at[q, r]], ssem[b],
                             add=True)
        return carry

    lax.fori_loop(0, NWIN, win, 0)
    wait_s((TCH - 1) % 2)
    plsc.subcore_barrier()
    pltpu.sync_copy(acc.at[pl.ds(base, RPT)], out.at[c, pl.ds(base, RPT)])


R = 1000


def _dinv_of(dp_ref):
    deg = dp_ref[0, :, 0] + dp_ref[1, :, 0] + 1.0
    return lax.rsqrt(deg)


def _tc1_body(x_ref, w_ref, dp_ref, g_ref, t_ref):
    dinv = _dinv_of(dp_ref)
    t = jnp.dot(x_ref[...], w_ref[...], preferred_element_type=jnp.float32)
    t_ref[...] = t
    g_ref[...] = t * dinv[:, None]


def _tc2_body(s_ref, dp_ref, t1_ref, w_ref, b_ref, t2_ref, g2_ref):
    dinv = _dinv_of(dp_ref)
    h = ((s_ref[0] + s_ref[1]) * dinv[:, None]
         + t1_ref[...] * (dinv * dinv)[:, None] + b_ref[...])
    t2 = jnp.dot(h, w_ref[...], preferred_element_type=jnp.float32)
    t2_ref[...] = t2
    g2_ref[...] = t2 * dinv[:, None]


def _tc3_body(s_ref, dp_ref, t2_ref, b_ref, out_ref):
    dinv = _dinv_of(dp_ref)
    out_ref[...] = ((s_ref[0] + s_ref[1]) * dinv[:, None]
                    + t2_ref[...] * (dinv * dinv)[:, None] + b_ref[...])


_row_spec = pl.BlockSpec((R, D), lambda i: (i, 0))
_w_spec = pl.BlockSpec((D, D), lambda i: (0, 0))
_s_spec = pl.BlockSpec((NC, R, D), lambda i: (0, i, 0))
_b_spec = pl.BlockSpec((1, D), lambda i: (0, 0))

_tc1 = pl.pallas_call(
    _tc1_body,
    grid=(N // R,),
    in_specs=[_row_spec, _w_spec, _s_spec],
    out_specs=[_row_spec, _row_spec],
    out_shape=[jax.ShapeDtypeStruct((N, D), jnp.float32)] * 2,
)

_tc2 = pl.pallas_call(
    _tc2_body,
    grid=(N // R,),
    in_specs=[_s_spec, _s_spec, _row_spec, _w_spec, _b_spec],
    out_specs=[_row_spec, _row_spec],
    out_shape=[jax.ShapeDtypeStruct((N, D), jnp.float32)] * 2,
)

_tc3 = pl.pallas_call(
    _tc3_body,
    grid=(N // R,),
    in_specs=[_s_spec, _s_spec, _row_spec, _b_spec],
    out_specs=_row_spec,
    out_shape=jax.ShapeDtypeStruct((N, D), jnp.float32),
)


def kernel(x, adj, W1, b1, W2, b2):
    src = adj[0].astype(jnp.int32)
    dst = adj[1].astype(jnp.int32)
    pad = EP - E
    src_p = jnp.concatenate([src, jnp.zeros((pad,), jnp.int32)])
    dst_p = jnp.concatenate([dst, jnp.full((pad,), N, jnp.int32)])
    src4 = src_p.reshape(NW, NWIN, W, CHUNK)
    dst4 = dst_p.reshape(NW, NWIN, W, CHUNK)
    dst3 = dst_p.reshape(NW, TCH, CHUNK)

    dp = _deg_kernel(dst3)
    g1, t1 = _tc1(x, W1, dp)
    s1 = _scatter_kernel(g1, src4, dst4)
    t2, g2 = _tc2(s1, dp, t1, W2, b1.reshape(1, D))
    s2 = _scatter_kernel(g2, src4, dst4)
    out = _tc3(s2, dp, t2, b2.reshape(1, D))
    return out


# speedup vs baseline: 1.7950x; 1.0767x over previous
import functools

import jax
import jax.numpy as jnp
from jax import lax
from jax.experimental import pallas as pl
from jax.experimental.pallas import tpu as pltpu
from jax.experimental.pallas import tpu_sc as plsc

N = 10000
E = 320000
D = 128

NC = 2
NS = 16
NW = NC * NS
CHUNK = 128
EP = 327680
TCH = EP // (NW * CHUNK)
W = 8
NWIN = TCH // W
T0 = 120
T1 = 40
NWCHUNKS = EP // CHUNK
NPAD = 10112
RPT = NPAD // NS

_mesh = plsc.VectorSubcoreMesh(core_axis_name="c", subcore_axis_name="s")


def _zero_slice(buf, acc, base, nrows):
    for k in range(nrows // CHUNK):
        pltpu.sync_copy(buf, acc.at[pl.ds(base + k * CHUNK, CHUNK)])
    rem = nrows % CHUNK
    if rem:
        pltpu.sync_copy(buf.at[pl.ds(0, rem)],
                        acc.at[pl.ds(base + nrows - rem, rem)])


def _fill(buf, val):
    def row(i, carry):
        for k in range(D // 16):
            buf[i, pl.ds(k * 16, 16)] = jnp.full((16,), val, jnp.float32)
        return carry
    lax.fori_loop(0, CHUNK, row, 0)


@functools.partial(
    pl.kernel,
    out_type=jax.ShapeDtypeStruct((NC, NPAD, D), jnp.float32),
    mesh=_mesh,
    scratch_types=[
        pltpu.VMEM((TCH, CHUNK), jnp.int32),
        pltpu.VMEM((CHUNK, D), jnp.float32),
        pltpu.VMEM_SHARED((NPAD, D), jnp.float32),
    ],
)
def _deg_kernel(dst3, out, dst_v, buf, dacc):
    c = lax.axis_index("c")
    s = lax.axis_index("s")
    w = c * NS + s
    pltpu.sync_copy(dst3.at[w], dst_v)
    _fill(buf, 0.0)
    base = s * RPT
    _zero_slice(buf, dacc, base, RPT)
    _fill(buf, 1.0)
    plsc.subcore_barrier()

    def chunk(j, carry):
        pltpu.sync_copy(buf, dacc.at[dst_v.at[j]], add=True)
        return carry
    lax.fori_loop(0, TCH, chunk, 0)
    plsc.subcore_barrier()
    pltpu.sync_copy(dacc.at[pl.ds(base, RPT)], out.at[c, pl.ds(base, RPT)])


@functools.partial(
    pl.kernel,
    out_type=jax.ShapeDtypeStruct((NC, NPAD, D), jnp.float32),
    mesh=_mesh,
    scratch_types=[
        pltpu.VMEM((2, W, CHUNK), jnp.int32),
        pltpu.VMEM((2, W, CHUNK), jnp.int32),
        [pltpu.VMEM((CHUNK, D), jnp.float32)] * 2,
        [pltpu.SemaphoreType.DMA] * 2,
        [pltpu.SemaphoreType.DMA] * 2,
        pltpu.SemaphoreType.DMA,
        pltpu.VMEM_SHARED((NPAD, D), jnp.float32),
    ],
)
def _scatter_kernel(g, src3w, dst3w, out, src_w, dst_w, bufs, gsem, ssem,
                    isem, acc):
    c = lax.axis_index("c")
    s = lax.axis_index("s")
    _fill(bufs[0], 0.0)
    base = s * RPT
    _zero_slice(bufs[0], acc, base, RPT)

    nch = jnp.where(c == 0, T0, T1)
    nwin_t = jnp.where(c == 0, T0 // W, T1 // W)
    basew = jnp.where(c == 0, s * (T0 // W),
                      NS * (T0 // W) + s * (T1 // W))

    pltpu.sync_copy(src3w.at[basew], src_w.at[0])
    pltpu.sync_copy(dst3w.at[basew], dst_w.at[0])
    plsc.subcore_barrier()

    def fire(idx_row, b):
        pltpu.async_copy(g.at[idx_row], bufs[b], gsem[b])

    def wait_g(b):
        pltpu.make_async_copy(g.at[pl.ds(0, CHUNK)], bufs[b], gsem[b]).wait()

    def wait_s(b):
        pltpu.make_async_copy(bufs[b], acc.at[dst_w.at[0, 0]], ssem[b]).wait()

    def wait_i():
        pltpu.make_async_copy(src3w.at[0], src_w.at[0], isem).wait()
        pltpu.make_async_copy(dst3w.at[0], dst_w.at[0], isem).wait()

    fire(src_w.at[0, 0], 0)

    def win(w, carry):
        q = lax.rem(w, 2)
        qn = lax.rem(w + 1, 2)
        for r in range(W):
            j = w * W + r
            b = r % 2
            bn = (r + 1) % 2

            @pl.when(j >= 1)
            def _():
                wait_s(bn)

            if r == 1:
                @pl.when(w + 1 < nwin_t)
                def _():
                    pltpu.async_copy(src3w.at[basew + w + 1], src_w.at[qn],
                                     isem)
                    pltpu.async_copy(dst3w.at[basew + w + 1], dst_w.at[qn],
                                     isem)

            if r == W - 2:
                @pl.when(w + 1 < nwin_t)
                def _():
                    wait_i()

            @pl.when(j + 1 < nch)
            def _():
                if r < W - 1:
                    fire(src_w.at[q, r + 1], bn)
                else:
                    fire(src_w.at[qn, 0], bn)

            wait_g(b)
            pltpu.async_copy(bufs[b], acc.at[dst_w.at[q, r]], ssem[b],
                             add=True)
        return carry

    lax.fori_loop(0, nwin_t, win, 0)
    wait_s(1)
    plsc.subcore_barrier()
    pltpu.sync_copy(acc.at[pl.ds(base, RPT)], out.at[c, pl.ds(base, RPT)])


R = 1000


def _dinv_of(dp_ref):
    deg = dp_ref[0, :, 0] + dp_ref[1, :, 0] + 1.0
    return lax.rsqrt(deg)


def _tc1_body(x_ref, w_ref, dp_ref, g_ref, t_ref):
    dinv = _dinv_of(dp_ref)
    t = jnp.dot(x_ref[...], w_ref[...], preferred_element_type=jnp.float32)
    t_ref[...] = t
    g_ref[...] = t * dinv[:, None]


def _tc2_body(s_ref, dp_ref, t1_ref, w_ref, b_ref, t2_ref, g2_ref):
    dinv = _dinv_of(dp_ref)
    h = ((s_ref[0] + s_ref[1]) * dinv[:, None]
         + t1_ref[...] * (dinv * dinv)[:, None] + b_ref[...])
    t2 = jnp.dot(h, w_ref[...], preferred_element_type=jnp.float32)
    t2_ref[...] = t2
    g2_ref[...] = t2 * dinv[:, None]


def _tc3_body(s_ref, dp_ref, t2_ref, b_ref, out_ref):
    dinv = _dinv_of(dp_ref)
    out_ref[...] = ((s_ref[0] + s_ref[1]) * dinv[:, None]
                    + t2_ref[...] * (dinv * dinv)[:, None] + b_ref[...])


_row_spec = pl.BlockSpec((R, D), lambda i: (i, 0))
_w_spec = pl.BlockSpec((D, D), lambda i: (0, 0))
_s_spec = pl.BlockSpec((NC, R, D), lambda i: (0, i, 0))
_b_spec = pl.BlockSpec((1, D), lambda i: (0, 0))

_tc1 = pl.pallas_call(
    _tc1_body,
    grid=(N // R,),
    in_specs=[_row_spec, _w_spec, _s_spec],
    out_specs=[_row_spec, _row_spec],
    out_shape=[jax.ShapeDtypeStruct((N, D), jnp.float32)] * 2,
)

_tc2 = pl.pallas_call(
    _tc2_body,
    grid=(N // R,),
    in_specs=[_s_spec, _s_spec, _row_spec, _w_spec, _b_spec],
    out_specs=[_row_spec, _row_spec],
    out_shape=[jax.ShapeDtypeStruct((N, D), jnp.float32)] * 2,
)

_tc3 = pl.pallas_call(
    _tc3_body,
    grid=(N // R,),
    in_specs=[_s_spec, _s_spec, _row_spec, _b_spec],
    out_specs=_row_spec,
    out_shape=jax.ShapeDtypeStruct((N, D), jnp.float32),
)


def kernel(x, adj, W1, b1, W2, b2):
    src = adj[0].astype(jnp.int32)
    dst = adj[1].astype(jnp.int32)
    pad = EP - E
    src_p = jnp.concatenate([src, jnp.zeros((pad,), jnp.int32)])
    dst_p = jnp.concatenate([dst, jnp.full((pad,), N, jnp.int32)])
    src3w = src_p.reshape(NWCHUNKS // W, W, CHUNK)
    dst3w = dst_p.reshape(NWCHUNKS // W, W, CHUNK)
    dst3 = dst_p.reshape(NW, TCH, CHUNK)

    dp = _deg_kernel(dst3)
    g1, t1 = _tc1(x, W1, dp)
    s1 = _scatter_kernel(g1, src3w, dst3w)
    t2, g2 = _tc2(s1, dp, t1, W2, b1.reshape(1, D))
    s2 = _scatter_kernel(g2, src3w, dst3w)
    out = _tc3(s2, dp, t2, b2.reshape(1, D))
    return out
